# baseline (speedup 1.0000x reference)
import jax
import jax.numpy as jnp
from jax.experimental import pallas as pl
from jax.experimental.pallas import tpu as pltpu

_Q, _D_IN, _D, _S, _V, _TOPK = 1024, 512, 256, 500, 64, 4
_BIG = 3.0e38
_TDIMS = (((1,), (1,)), ((), ()))


def _conf_kernel(sample_ref, w_ref, keys_ref, k2_ref,
                 mean_ref, topk_ref, min_ref, ml_ref, tl_ref, nl_ref,
                 vecm2_s, q2_s, m1_s, m2_s, m3_s, m4_s, wacc,
                 c0, c1, c2, c3, c4, c5, c6):
    r = pl.program_id(0)
    j = pl.program_id(1)
    classes = (c0, c1, c2, c3, c4, c5, c6)

    @pl.when(jnp.logical_and(r == 0, j == 0))
    def _init():
        enc = jnp.tanh(jnp.dot(sample_ref[...], w_ref[...],
                               preferred_element_type=jnp.float32))
        e2 = enc * enc
        acc = e2[:, 0:8]
        for g in range(1, 32):
            acc = acc + e2[:, 8 * g:8 * g + 8]
        t = acc[:, 0:4] + acc[:, 4:8]
        t = t[:, 0:2] + t[:, 2:4]
        q2 = t[:, 0:1] + t[:, 1:2]
        q2_s[...] = jnp.broadcast_to(q2, q2_s.shape)
        vecm2_s[...] = -2.0 * enc
        big = jnp.full(m1_s.shape, _BIG, jnp.float32)
        m1_s[...] = big
        m2_s[...] = big
        m3_s[...] = big
        m4_s[...] = big

    keys = keys_ref[:, 0, 0, :]
    dotm2 = jax.lax.dot_general(vecm2_s[...], keys, _TDIMS,
                                preferred_element_type=jnp.float32)
    dist = (q2_s[:, 0:1] + k2_ref[0]) + dotm2

    @pl.when(j == 0)
    def _acc_first():
        wacc[...] = dist

    @pl.when(j > 0)
    def _acc_rest():
        wacc[...] = wacc[...] + dist

    @pl.when(jnp.logical_and(j == 7, r < 7))
    def _acc_store():
        done = wacc[...]
        for i, c in enumerate(classes):
            @pl.when(r == i)
            def _store(c=c, done=done):
                c[...] = done

    x = dist
    m1 = m1_s[...]
    m1_s[...] = jnp.minimum(m1, x)
    x = jnp.maximum(m1, x)
    m2 = m2_s[...]
    m2_s[...] = jnp.minimum(m2, x)
    x = jnp.maximum(m2, x)
    m3 = m3_s[...]
    m3_s[...] = jnp.minimum(m3, x)
    x = jnp.maximum(m3, x)
    m4_s[...] = jnp.minimum(m4_s[...], x)

    @pl.when(jnp.logical_and(r == 7, j == 7))
    def _finalize():
        a7 = wacc[...]
        b0 = c0[...] + c4[...]
        b1 = c1[...] + c5[...]
        b2 = c2[...] + c6[...]
        b3 = c3[...] + a7
        total = (b0 + b2) + (b1 + b3)
        mean = total * (1.0 / _V)
        m1v = m1_s[...]
        topk = ((m1v + m3_s[...]) + (m2_s[...] + m4_s[...])) * (1.0 / _TOPK)
        mean_ref[...] = mean
        topk_ref[...] = topk
        min_ref[...] = m1v
        ml = jnp.argmin(mean, axis=1).astype(jnp.int32)
        tl = jnp.argmin(topk, axis=1).astype(jnp.int32)
        nl = jnp.argmin(m1v, axis=1).astype(jnp.int32)
        ml_ref[...] = jnp.broadcast_to(ml[:, None], ml_ref.shape)
        tl_ref[...] = jnp.broadcast_to(tl[:, None], tl_ref.shape)
        nl_ref[...] = jnp.broadcast_to(nl[:, None], nl_ref.shape)


@jax.jit
def kernel(sample, W_enc, speaker_vectors):
    keys4d = jnp.reshape(speaker_vectors, (_S, _V, 1, _D))
    k2 = jnp.sum(speaker_vectors * speaker_vectors, axis=2)
    k2v = jnp.transpose(k2)[:, None, :]
    f32 = jnp.float32
    out = pl.pallas_call(
        _conf_kernel,
        grid=(8, 8),
        in_specs=[
            pl.BlockSpec((_Q, _D_IN), lambda r, j: (0, 0)),
            pl.BlockSpec((_D_IN, _D), lambda r, j: (0, 0)),
            pl.BlockSpec((_S, 1, 1, _D), lambda r, j: (0, 8 * j + r, 0, 0)),
            pl.BlockSpec((1, 1, _S), lambda r, j: (8 * j + r, 0, 0)),
        ],
        out_specs=[
            pl.BlockSpec((_Q, _S), lambda r, j: (0, 0)),
            pl.BlockSpec((_Q, _S), lambda r, j: (0, 0)),
            pl.BlockSpec((_Q, _S), lambda r, j: (0, 0)),
            pl.BlockSpec((_Q, 128), lambda r, j: (0, 0)),
            pl.BlockSpec((_Q, 128), lambda r, j: (0, 0)),
            pl.BlockSpec((_Q, 128), lambda r, j: (0, 0)),
        ],
        out_shape=[
            jax.ShapeDtypeStruct((_Q, _S), f32),
            jax.ShapeDtypeStruct((_Q, _S), f32),
            jax.ShapeDtypeStruct((_Q, _S), f32),
            jax.ShapeDtypeStruct((_Q, 128), jnp.int32),
            jax.ShapeDtypeStruct((_Q, 128), jnp.int32),
            jax.ShapeDtypeStruct((_Q, 128), jnp.int32),
        ],
        scratch_shapes=[
            pltpu.VMEM((_Q, _D), f32),
            pltpu.VMEM((_Q, 128), f32),
            pltpu.VMEM((_Q, _S), f32),
            pltpu.VMEM((_Q, _S), f32),
            pltpu.VMEM((_Q, _S), f32),
            pltpu.VMEM((_Q, _S), f32),
        ] + [pltpu.VMEM((_Q, _S), f32) for _ in range(8)],
        compiler_params=pltpu.CompilerParams(
            dimension_semantics=("arbitrary", "arbitrary")),
    )(sample, W_enc, keys4d, k2v)
    mean_o, topk_o, min_o, ml_o, tl_o, nl_o = out
    stats = jnp.stack([mean_o, topk_o, min_o], axis=-1)
    return stats, ml_o[:, 0], tl_o[:, 0], nl_o[:, 0]


# speedup vs baseline: 1.1869x; 1.1869x over previous
import jax
import jax.numpy as jnp
from jax.experimental import pallas as pl
from jax.experimental.pallas import tpu as pltpu

_Q, _D_IN, _D, _S, _V, _TOPK = 1024, 512, 256, 500, 64, 4
_BIG = 3.0e38
_TDIMS = (((1,), (1,)), ((), ()))


def _conf_kernel(sample_ref, w_ref, keys_a_ref, keys_b_ref, k2a_ref, k2b_ref,
                 mean_ref, topk_ref, min_ref, ml_ref, tl_ref, nl_ref,
                 vecm2_s, q2_s, m1_s, m2_s, m3_s, m4_s, wacc,
                 c0, c1, c2, c3):
    r = pl.program_id(0)
    t = pl.program_id(1)
    classes = (c0, c1, c2, c3)

    @pl.when(jnp.logical_and(r == 0, t == 0))
    def _init():
        enc = jnp.tanh(jnp.dot(sample_ref[...], w_ref[...],
                               preferred_element_type=jnp.float32))
        e2 = enc * enc
        acc = e2[:, 0:8]
        for g in range(1, 32):
            acc = acc + e2[:, 8 * g:8 * g + 8]
        t = acc[:, 0:4] + acc[:, 4:8]
        t = t[:, 0:2] + t[:, 2:4]
        q2 = t[:, 0:1] + t[:, 1:2]
        q2_s[...] = jnp.broadcast_to(q2, q2_s.shape)
        vecm2_s[...] = -2.0 * enc
        big = jnp.full(m1_s.shape, _BIG, jnp.float32)
        m1_s[...] = big
        m2_s[...] = big
        m3_s[...] = big
        m4_s[...] = big

    vecm2 = vecm2_s[...]
    q2k = q2_s[:, 0:1]
    keys_a = keys_a_ref[:, 0, 0, :]
    keys_b = keys_b_ref[:, 0, 0, :]
    dotm2_a = jax.lax.dot_general(vecm2, keys_a, _TDIMS,
                                  preferred_element_type=jnp.float32)
    dotm2_b = jax.lax.dot_general(vecm2, keys_b, _TDIMS,
                                  preferred_element_type=jnp.float32)
    dist_a = (q2k + k2a_ref[0]) + dotm2_a
    dist_b = (q2k + k2b_ref[0]) + dotm2_b

    @pl.when(t == 0)
    def _acc_first():
        wacc[...] = dist_a + dist_b

    @pl.when(t > 0)
    def _acc_rest():
        wacc[...] = (wacc[...] + dist_a) + dist_b

    @pl.when(jnp.logical_and(t == 3, r < 7))
    def _acc_store():
        done = wacc[...]
        for i, c in enumerate(classes):
            @pl.when(r == i)
            def _store(c=c, done=done):
                c[...] = done
        for i, c in enumerate(classes[:3]):
            @pl.when(r == i + 4)
            def _fold(c=c, done=done):
                c[...] = c[...] + done

    xa = dist_a
    xb = dist_b
    m1 = m1_s[...]
    m1n = jnp.minimum(m1, xa)
    xa = jnp.maximum(m1, xa)
    m1_s[...] = jnp.minimum(m1n, xb)
    xb = jnp.maximum(m1n, xb)
    m2 = m2_s[...]
    m2n = jnp.minimum(m2, xa)
    xa = jnp.maximum(m2, xa)
    m2_s[...] = jnp.minimum(m2n, xb)
    xb = jnp.maximum(m2n, xb)
    m3 = m3_s[...]
    m3n = jnp.minimum(m3, xa)
    xa = jnp.maximum(m3, xa)
    m3_s[...] = jnp.minimum(m3n, xb)
    xb = jnp.maximum(m3n, xb)
    m4 = m4_s[...]
    m4n = jnp.minimum(m4, xa)
    m4_s[...] = jnp.minimum(m4n, xb)

    @pl.when(jnp.logical_and(r == 7, t == 3))
    def _finalize():
        b3 = c3[...] + wacc[...]
        total = (c0[...] + c2[...]) + (c1[...] + b3)
        mean = total * (1.0 / _V)
        m1v = m1_s[...]
        topk = ((m1v + m3_s[...]) + (m2_s[...] + m4_s[...])) * (1.0 / _TOPK)
        mean_ref[...] = mean
        topk_ref[...] = topk
        min_ref[...] = m1v
        ml = jnp.argmin(mean, axis=1).astype(jnp.int32)
        tl = jnp.argmin(topk, axis=1).astype(jnp.int32)
        nl = jnp.argmin(m1v, axis=1).astype(jnp.int32)
        ml_ref[...] = jnp.broadcast_to(ml[:, None], ml_ref.shape)
        tl_ref[...] = jnp.broadcast_to(tl[:, None], tl_ref.shape)
        nl_ref[...] = jnp.broadcast_to(nl[:, None], nl_ref.shape)


@jax.jit
def kernel(sample, W_enc, speaker_vectors):
    keys4d = jnp.reshape(speaker_vectors, (_S, _V, 1, _D))
    k2 = jnp.sum(speaker_vectors * speaker_vectors, axis=2)
    k2v = jnp.transpose(k2)[:, None, :]
    f32 = jnp.float32
    out = pl.pallas_call(
        _conf_kernel,
        grid=(8, 4),
        in_specs=[
            pl.BlockSpec((_Q, _D_IN), lambda r, t: (0, 0)),
            pl.BlockSpec((_D_IN, _D), lambda r, t: (0, 0)),
            pl.BlockSpec((_S, 1, 1, _D), lambda r, t: (0, 16 * t + r, 0, 0)),
            pl.BlockSpec((_S, 1, 1, _D),
                         lambda r, t: (0, 16 * t + 8 + r, 0, 0)),
            pl.BlockSpec((1, 1, _S), lambda r, t: (16 * t + r, 0, 0)),
            pl.BlockSpec((1, 1, _S), lambda r, t: (16 * t + 8 + r, 0, 0)),
        ],
        out_specs=[
            pl.BlockSpec((_Q, _S), lambda r, t: (0, 0)),
            pl.BlockSpec((_Q, _S), lambda r, t: (0, 0)),
            pl.BlockSpec((_Q, _S), lambda r, t: (0, 0)),
            pl.BlockSpec((_Q, 128), lambda r, t: (0, 0)),
            pl.BlockSpec((_Q, 128), lambda r, t: (0, 0)),
            pl.BlockSpec((_Q, 128), lambda r, t: (0, 0)),
        ],
        out_shape=[
            jax.ShapeDtypeStruct((_Q, _S), f32),
            jax.ShapeDtypeStruct((_Q, _S), f32),
            jax.ShapeDtypeStruct((_Q, _S), f32),
            jax.ShapeDtypeStruct((_Q, 128), jnp.int32),
            jax.ShapeDtypeStruct((_Q, 128), jnp.int32),
            jax.ShapeDtypeStruct((_Q, 128), jnp.int32),
        ],
        scratch_shapes=[
            pltpu.VMEM((_Q, _D), f32),
            pltpu.VMEM((_Q, 128), f32),
            pltpu.VMEM((_Q, _S), f32),
            pltpu.VMEM((_Q, _S), f32),
            pltpu.VMEM((_Q, _S), f32),
            pltpu.VMEM((_Q, _S), f32),
        ] + [pltpu.VMEM((_Q, _S), f32) for _ in range(5)],
        compiler_params=pltpu.CompilerParams(
            dimension_semantics=("arbitrary", "arbitrary")),
    )(sample, W_enc, keys4d, keys4d, k2v, k2v)
    mean_o, topk_o, min_o, ml_o, tl_o, nl_o = out
    stats = jnp.stack([mean_o, topk_o, min_o], axis=-1)
    return stats, ml_o[:, 0], tl_o[:, 0], nl_o[:, 0]


# speedup vs baseline: 1.2811x; 1.0793x over previous
import jax
import jax.numpy as jnp
from jax.experimental import pallas as pl
from jax.experimental.pallas import tpu as pltpu

_Q, _D_IN, _D, _S, _V, _TOPK = 1024, 512, 256, 500, 64, 4
_BIG = 3.0e38
_TDIMS = (((1,), (1,)), ((), ()))


def _conf_kernel(sample_ref, w_ref, keys_a_ref, keys_b_ref, keys_c_ref,
                 keys_d_ref, k2a_ref, k2b_ref, k2c_ref, k2d_ref,
                 mean_ref, topk_ref, min_ref, ml_ref, tl_ref, nl_ref,
                 vecm2_s, q2_s, m1_s, m2_s, m3_s, m4_s, wacc,
                 c0, c1, c2, c3):
    r = pl.program_id(0)
    t = pl.program_id(1)
    classes = (c0, c1, c2, c3)

    @pl.when(jnp.logical_and(r == 0, t == 0))
    def _init():
        enc = jnp.tanh(jnp.dot(sample_ref[...], w_ref[...],
                               preferred_element_type=jnp.float32))
        e2 = enc * enc
        acc = e2[:, 0:8]
        for g in range(1, 32):
            acc = acc + e2[:, 8 * g:8 * g + 8]
        t = acc[:, 0:4] + acc[:, 4:8]
        t = t[:, 0:2] + t[:, 2:4]
        q2 = t[:, 0:1] + t[:, 1:2]
        q2_s[...] = jnp.broadcast_to(q2, q2_s.shape)
        vecm2_s[...] = -2.0 * enc
        big = jnp.full(m1_s.shape, _BIG, jnp.float32)
        m1_s[...] = big
        m2_s[...] = big
        m3_s[...] = big
        m4_s[...] = big

    vecm2 = vecm2_s[...]
    q2k = q2_s[:, 0:1]
    dists = []
    for kref, k2ref in ((keys_a_ref, k2a_ref), (keys_b_ref, k2b_ref),
                        (keys_c_ref, k2c_ref), (keys_d_ref, k2d_ref)):
        keys = kref[:, 0, 0, :]
        dotm2 = jax.lax.dot_general(vecm2, keys, _TDIMS,
                                    preferred_element_type=jnp.float32)
        dists.append((q2k + k2ref[0]) + dotm2)

    @pl.when(t == 0)
    def _acc_first():
        wacc[...] = ((dists[0] + dists[1]) + dists[2]) + dists[3]

    @pl.when(t > 0)
    def _acc_rest():
        acc = wacc[...]
        for dv in dists:
            acc = acc + dv
        wacc[...] = acc

    @pl.when(jnp.logical_and(t == 1, r < 7))
    def _acc_store():
        done = wacc[...]
        for i, c in enumerate(classes):
            @pl.when(r == i)
            def _store(c=c, done=done):
                c[...] = done
        for i, c in enumerate(classes[:3]):
            @pl.when(r == i + 4)
            def _fold(c=c, done=done):
                c[...] = c[...] + done

    xs = list(dists)
    for m_s in (m1_s, m2_s, m3_s):
        m = m_s[...]
        nxt = []
        for x in xs:
            mn = jnp.minimum(m, x)
            nxt.append(jnp.maximum(m, x))
            m = mn
        m_s[...] = m
        xs = nxt
    m = m4_s[...]
    for x in xs:
        m = jnp.minimum(m, x)
    m4_s[...] = m

    @pl.when(jnp.logical_and(r == 7, t == 1))
    def _finalize():
        b3 = c3[...] + wacc[...]
        total = (c0[...] + c2[...]) + (c1[...] + b3)
        mean = total * (1.0 / _V)
        m1v = m1_s[...]
        topk = ((m1v + m3_s[...]) + (m2_s[...] + m4_s[...])) * (1.0 / _TOPK)
        mean_ref[...] = mean
        topk_ref[...] = topk
        min_ref[...] = m1v
        ml = jnp.argmin(mean, axis=1).astype(jnp.int32)
        tl = jnp.argmin(topk, axis=1).astype(jnp.int32)
        nl = jnp.argmin(m1v, axis=1).astype(jnp.int32)
        ml_ref[...] = jnp.broadcast_to(ml[:, None], ml_ref.shape)
        tl_ref[...] = jnp.broadcast_to(tl[:, None], tl_ref.shape)
        nl_ref[...] = jnp.broadcast_to(nl[:, None], nl_ref.shape)


@jax.jit
def kernel(sample, W_enc, speaker_vectors):
    keys4d = jnp.reshape(speaker_vectors, (_S, _V, 1, _D))
    k2 = jnp.sum(speaker_vectors * speaker_vectors, axis=2)
    k2v = jnp.transpose(k2)[:, None, :]
    f32 = jnp.float32
    out = pl.pallas_call(
        _conf_kernel,
        grid=(8, 2),
        in_specs=[
            pl.BlockSpec((_Q, _D_IN), lambda r, t: (0, 0)),
            pl.BlockSpec((_D_IN, _D), lambda r, t: (0, 0)),
        ] + [
            pl.BlockSpec((_S, 1, 1, _D),
                         lambda r, t, p=p: (0, 32 * t + 8 * p + r, 0, 0))
            for p in range(4)
        ] + [
            pl.BlockSpec((1, 1, _S),
                         lambda r, t, p=p: (32 * t + 8 * p + r, 0, 0))
            for p in range(4)
        ],
        out_specs=[
            pl.BlockSpec((_Q, _S), lambda r, t: (0, 0)),
            pl.BlockSpec((_Q, _S), lambda r, t: (0, 0)),
            pl.BlockSpec((_Q, _S), lambda r, t: (0, 0)),
            pl.BlockSpec((_Q, 128), lambda r, t: (0, 0)),
            pl.BlockSpec((_Q, 128), lambda r, t: (0, 0)),
            pl.BlockSpec((_Q, 128), lambda r, t: (0, 0)),
        ],
        out_shape=[
            jax.ShapeDtypeStruct((_Q, _S), f32),
            jax.ShapeDtypeStruct((_Q, _S), f32),
            jax.ShapeDtypeStruct((_Q, _S), f32),
            jax.ShapeDtypeStruct((_Q, 128), jnp.int32),
            jax.ShapeDtypeStruct((_Q, 128), jnp.int32),
            jax.ShapeDtypeStruct((_Q, 128), jnp.int32),
        ],
        scratch_shapes=[
            pltpu.VMEM((_Q, _D), f32),
            pltpu.VMEM((_Q, 128), f32),
            pltpu.VMEM((_Q, _S), f32),
            pltpu.VMEM((_Q, _S), f32),
            pltpu.VMEM((_Q, _S), f32),
            pltpu.VMEM((_Q, _S), f32),
        ] + [pltpu.VMEM((_Q, _S), f32) for _ in range(5)],
        compiler_params=pltpu.CompilerParams(
            dimension_semantics=("arbitrary", "arbitrary"),
            vmem_limit_bytes=100 * 1024 * 1024),
    )(sample, W_enc, keys4d, keys4d, keys4d, keys4d, k2v, k2v, k2v, k2v)
    mean_o, topk_o, min_o, ml_o, tl_o, nl_o = out
    stats = jnp.stack([mean_o, topk_o, min_o], axis=-1)
    return stats, ml_o[:, 0], tl_o[:, 0], nl_o[:, 0]
